# baseline (speedup 1.0000x reference)
import functools

import jax
import jax.numpy as jnp
from jax import lax
from jax.experimental import pallas as pl
from jax.experimental.pallas import tpu as pltpu
from jax.experimental.pallas import tpu_sc as plsc

B, T, V, D, DD, L, H = 4096, 26, 100000, 64, 128, 20, 512

NC, NS = 2, 16
NW = NC * NS
NB = B * T
BAGS_PER_W = NB // NW
CHUNK_BAGS = 32
IDX_PER_CHUNK = CHUNK_BAGS * L
IDX_ROWS = IDX_PER_CHUNK // 128
N_CHUNKS = BAGS_PER_W // CHUNK_BAGS


def _emb_bag_sc(tab_flat, idx_flat):
    mesh = plsc.VectorSubcoreMesh(core_axis_name="c", subcore_axis_name="s")

    @functools.partial(
        pl.kernel,
        mesh=mesh,
        out_type=jax.ShapeDtypeStruct((NB, D), jnp.float32),
        scratch_types=[
            pltpu.VMEM((IDX_PER_CHUNK,), jnp.int32),
            pltpu.VMEM((IDX_PER_CHUNK, D), jnp.float32),
            pltpu.VMEM((CHUNK_BAGS, D), jnp.float32),
            pltpu.SemaphoreType.DMA,
        ],
        compiler_params=pltpu.CompilerParams(use_tc_tiling_on_sc=False),
    )
    def emb_kernel(tab_hbm, idx_hbm, out_hbm, idx_v, rows_v, out_v, sem):
        wid = lax.axis_index("s") * NC + lax.axis_index("c")

        @pl.loop(0, N_CHUNKS)
        def _(c):
            bag0 = wid * BAGS_PER_W + c * CHUNK_BAGS
            pltpu.sync_copy(idx_hbm.at[pl.ds(bag0 * L, IDX_PER_CHUNK)], idx_v)
            handles = []
            for j in range(IDX_ROWS):
                handles.append(
                    pltpu.async_copy(
                        tab_hbm.at[idx_v.at[pl.ds(j * 128, 128)]],
                        rows_v.at[pl.ds(j * 128, 128)],
                        sem,
                    )
                )
            for h in handles:
                h.wait()

            @pl.loop(0, CHUNK_BAGS)
            def _(w):
                base = w * L
                for d in range(D // 16):
                    sl = pl.ds(d * 16, 16)

                    def body(i, acc):
                        return acc + rows_v[base + i, sl]

                    out_v[w, sl] = lax.fori_loop(
                        0, L, body, jnp.zeros((16,), jnp.float32)
                    )

            pltpu.sync_copy(out_v, out_hbm.at[pl.ds(bag0, CHUNK_BAGS)])

    return emb_kernel(tab_flat, idx_flat)


BLK = 512


def _mlp_tc(dense_features, emb_flat, dense_W1, dense_b1, dense_W2, dense_b2,
            W1d, W1e, out_b1, out_W2, out_b2):
    def body(df, emb, dW1, db1, dW2, db2, w1d, w1e, ob1, oW2, ob2, out):
        cdims = (((1,), (1,)), ((), ()))
        h1 = lax.dot_general(df[...], dW1[...], cdims,
                             preferred_element_type=jnp.float32)
        h1 = jnp.maximum(h1 + db1[...], 0.0)
        dx = lax.dot_general(h1, dW2[...], cdims,
                             preferred_element_type=jnp.float32)
        dx = jnp.maximum(dx + db2[...], 0.0)
        a = lax.dot_general(dx, w1d[...], cdims,
                            preferred_element_type=jnp.float32)
        w1e_all = w1e[...]
        for t in range(T):
            a = a + lax.dot_general(emb[t], w1e_all[:, t * D:(t + 1) * D],
                                    cdims, preferred_element_type=jnp.float32)
        h = jnp.maximum(a + ob1[...], 0.0)
        o = lax.dot_general(h, oW2[...], cdims,
                            preferred_element_type=jnp.float32)
        out[...] = jnp.maximum(o + ob2[0, 0], 0.0)

    F_E = T * D
    whole = lambda shape: pl.BlockSpec(shape, lambda i: (0, 0))
    whole3 = lambda shape: pl.BlockSpec(shape, lambda i: (0, 0, 0))
    return pl.pallas_call(
        body,
        grid=(B // BLK,),
        in_specs=[
            pl.BlockSpec((BLK, DD), lambda i: (i, 0)),
            pl.BlockSpec((T, BLK, D), lambda i: (0, i, 0)),
            whole((DD, DD)),
            whole((1, DD)),
            whole((D, DD)),
            whole((1, D)),
            whole((H, D)),
            whole((H, F_E)),
            whole((1, H)),
            whole((128, H)),
            whole((1, 1)),
        ],
        out_specs=pl.BlockSpec((BLK, 128), lambda i: (i, 0)),
        out_shape=jax.ShapeDtypeStruct((B, 128), jnp.float32),
    )(dense_features, emb_flat, dense_W1, dense_b1.reshape(1, DD),
      dense_W2, dense_b2.reshape(1, D), W1d, W1e, out_b1.reshape(1, H),
      jnp.zeros((128, H), jnp.float32).at[0].set(out_W2[0]),
      out_b2.reshape(1, 1))[:, :1]


def kernel(dense_features, sparse_features, dense_W1, dense_b1, dense_W2,
           dense_b2, tables, out_W1, out_b1, out_W2, out_b2):
    assert sparse_features.shape == (T, B, L)
    assert tables.shape == (T, V, D)

    idx = sparse_features.astype(jnp.int32)
    idx = idx + (jnp.arange(T, dtype=jnp.int32) * V)[:, None, None]
    idx_flat = idx.reshape(-1)

    tab_flat = tables.reshape(T * V, D)
    emb = _emb_bag_sc(tab_flat, idx_flat)
    emb3 = emb.reshape(T, B, D)

    W1d = out_W1[:, :D]
    W1e = out_W1[:, D:]
    return _mlp_tc(dense_features, emb3, dense_W1, dense_b1, dense_W2,
                   dense_b2, W1d, W1e, out_b1, out_W2, out_b2)


# speedup vs baseline: 1.1879x; 1.1879x over previous
import functools

import jax
import jax.numpy as jnp
from jax import lax
from jax.experimental import pallas as pl
from jax.experimental.pallas import tpu as pltpu
from jax.experimental.pallas import tpu_sc as plsc

B, T, V, D, DD, L, H = 4096, 26, 100000, 64, 128, 20, 512

NC, NS = 2, 16
NW = NC * NS
NB = B * T
CHUNK_BAGS = 32
IDX_PER_CHUNK = CHUNK_BAGS * L
IDX_ROWS = IDX_PER_CHUNK // 128
BAGS_PER_TW = B // NW
NCPT = BAGS_PER_TW // CHUNK_BAGS
N_CHUNKS = T * NCPT


def _emb_bag_sc(tables, idx_flat):
    mesh = plsc.VectorSubcoreMesh(core_axis_name="c", subcore_axis_name="s")

    @functools.partial(
        pl.kernel,
        mesh=mesh,
        out_type=jax.ShapeDtypeStruct((NB, D), jnp.float32),
        scratch_types=[
            pltpu.VMEM((IDX_PER_CHUNK,), jnp.int32),
            pltpu.VMEM((IDX_PER_CHUNK,), jnp.int32),
            pltpu.VMEM((IDX_PER_CHUNK, D), jnp.float32),
            pltpu.VMEM((IDX_PER_CHUNK, D), jnp.float32),
            pltpu.VMEM((CHUNK_BAGS, D), jnp.float32),
            pltpu.VMEM((CHUNK_BAGS, D), jnp.float32),
            pltpu.SemaphoreType.DMA,
            pltpu.SemaphoreType.DMA,
            pltpu.SemaphoreType.DMA,
            pltpu.SemaphoreType.DMA,
        ],
        compiler_params=pltpu.CompilerParams(use_tc_tiling_on_sc=False),
    )
    def emb_kernel(tab_hbm, idx_hbm, out_hbm,
                   idx_v0, idx_v1, rows_v0, rows_v1, out_v0, out_v1,
                   g0, g1, o0, o1):
        wid = lax.axis_index("s") * NC + lax.axis_index("c")

        def bag0_of(k):
            t = k // NCPT
            c = k % NCPT
            return t, t * B + wid * BAGS_PER_TW + c * CHUNK_BAGS

        def fire(k, idx_v, rows_v, gsem):
            t, bag0 = bag0_of(k)
            pltpu.sync_copy(idx_hbm.at[pl.ds(bag0 * L, IDX_PER_CHUNK)], idx_v)
            for j in range(IDX_ROWS):
                sl = pl.ds(j * 128, 128)
                pltpu.async_copy(
                    tab_hbm.at[t].at[idx_v.at[sl]], rows_v.at[sl], gsem
                )

        def drain_rows(rows_v, gsem):
            pltpu.make_async_copy(
                tab_hbm.at[0, pl.ds(0, IDX_PER_CHUNK)], rows_v, gsem
            ).wait()

        def accum_write(k, rows_v, out_v, osem):
            _, bag0 = bag0_of(k)

            @pl.when(k >= 2)
            def _():
                pltpu.make_async_copy(
                    out_hbm.at[pl.ds(0, CHUNK_BAGS)], out_v, osem
                ).wait()

            @pl.loop(0, CHUNK_BAGS)
            def _(w):
                base = w * L
                for d in range(D // 16):
                    sl = pl.ds(d * 16, 16)
                    a0 = rows_v[base, sl]
                    a1 = rows_v[base + 1, sl]
                    for i in range(2, L, 2):
                        a0 = a0 + rows_v[base + i, sl]
                        a1 = a1 + rows_v[base + i + 1, sl]
                    out_v[w, sl] = a0 + a1

            pltpu.async_copy(out_v, out_hbm.at[pl.ds(bag0, CHUNK_BAGS)], osem)

        fire(0, idx_v0, rows_v0, g0)

        @pl.loop(0, N_CHUNKS - 1)
        def _(k):
            @pl.when(k % 2 == 0)
            def _():
                fire(k + 1, idx_v1, rows_v1, g1)
                drain_rows(rows_v0, g0)
                accum_write(k, rows_v0, out_v0, o0)

            @pl.when(k % 2 == 1)
            def _():
                fire(k + 1, idx_v0, rows_v0, g0)
                drain_rows(rows_v1, g1)
                accum_write(k, rows_v1, out_v1, o1)

        drain_rows(rows_v1, g1)
        accum_write(N_CHUNKS - 1, rows_v1, out_v1, o1)

        pltpu.make_async_copy(
            out_hbm.at[pl.ds(0, CHUNK_BAGS)], out_v0, o0
        ).wait()
        pltpu.make_async_copy(
            out_hbm.at[pl.ds(0, CHUNK_BAGS)], out_v1, o1
        ).wait()

    return emb_kernel(tables, idx_flat)


BLK = 512


def _mlp_tc(dense_features, emb_flat, dense_W1, dense_b1, dense_W2, dense_b2,
            W1d, W1e, out_b1, out_W2, out_b2):
    def body(df, emb, dW1, db1, dW2, db2, w1d, w1e, ob1, oW2, ob2, out):
        cdims = (((1,), (1,)), ((), ()))
        h1 = lax.dot_general(df[...], dW1[...], cdims,
                             preferred_element_type=jnp.float32)
        h1 = jnp.maximum(h1 + db1[...], 0.0)
        dx = lax.dot_general(h1, dW2[...], cdims,
                             preferred_element_type=jnp.float32)
        dx = jnp.maximum(dx + db2[...], 0.0)
        a = lax.dot_general(dx, w1d[...], cdims,
                            preferred_element_type=jnp.float32)
        w1e_all = w1e[...]
        for t in range(T):
            a = a + lax.dot_general(emb[t], w1e_all[:, t * D:(t + 1) * D],
                                    cdims, preferred_element_type=jnp.float32)
        h = jnp.maximum(a + ob1[...], 0.0)
        o = lax.dot_general(h, oW2[...], cdims,
                            preferred_element_type=jnp.float32)
        out[...] = jnp.maximum(o + ob2[0, 0], 0.0)

    F_E = T * D
    whole = lambda shape: pl.BlockSpec(shape, lambda i: (0, 0))
    return pl.pallas_call(
        body,
        grid=(B // BLK,),
        in_specs=[
            pl.BlockSpec((BLK, DD), lambda i: (i, 0)),
            pl.BlockSpec((T, BLK, D), lambda i: (0, i, 0)),
            whole((DD, DD)),
            whole((1, DD)),
            whole((D, DD)),
            whole((1, D)),
            whole((H, D)),
            whole((H, F_E)),
            whole((1, H)),
            whole((128, H)),
            whole((1, 1)),
        ],
        out_specs=pl.BlockSpec((BLK, 128), lambda i: (i, 0)),
        out_shape=jax.ShapeDtypeStruct((B, 128), jnp.float32),
    )(dense_features, emb_flat, dense_W1, dense_b1.reshape(1, DD),
      dense_W2, dense_b2.reshape(1, D), W1d, W1e, out_b1.reshape(1, H),
      jnp.zeros((128, H), jnp.float32).at[0].set(out_W2[0]),
      out_b2.reshape(1, 1))[:, :1]


def kernel(dense_features, sparse_features, dense_W1, dense_b1, dense_W2,
           dense_b2, tables, out_W1, out_b1, out_W2, out_b2):
    assert sparse_features.shape == (T, B, L)
    assert tables.shape == (T, V, D)

    idx_flat = sparse_features.astype(jnp.int32).reshape(-1)

    emb = _emb_bag_sc(tables, idx_flat)
    emb3 = emb.reshape(T, B, D)

    W1d = out_W1[:, :D]
    W1e = out_W1[:, D:]
    return _mlp_tc(dense_features, emb3, dense_W1, dense_b1, dense_W2,
                   dense_b2, W1d, W1e, out_b1, out_W2, out_b2)


# speedup vs baseline: 1.2090x; 1.0177x over previous
import functools

import jax
import jax.numpy as jnp
from jax import lax
from jax.experimental import pallas as pl
from jax.experimental.pallas import tpu as pltpu
from jax.experimental.pallas import tpu_sc as plsc

B, T, V, D, DD, L, H = 4096, 26, 100000, 64, 128, 20, 512

NC, NS = 2, 16
NW = NC * NS
CHUNK_BAGS = 32
IDX_PER_CHUNK = CHUNK_BAGS * L
IDX_ROWS = IDX_PER_CHUNK // 128


def _emb_bag_sc(tables, idx_flat, nb):
    bsub = nb // T
    BAGS_PER_TW = bsub // NW
    NCPT = BAGS_PER_TW // CHUNK_BAGS
    N_CHUNKS = T * NCPT
    NB = nb
    mesh = plsc.VectorSubcoreMesh(core_axis_name="c", subcore_axis_name="s")

    @functools.partial(
        pl.kernel,
        mesh=mesh,
        out_type=jax.ShapeDtypeStruct((NB, D), jnp.float32),
        scratch_types=[
            pltpu.VMEM((IDX_PER_CHUNK,), jnp.int32),
            pltpu.VMEM((IDX_PER_CHUNK,), jnp.int32),
            pltpu.VMEM((IDX_PER_CHUNK, D), jnp.float32),
            pltpu.VMEM((IDX_PER_CHUNK, D), jnp.float32),
            pltpu.VMEM((CHUNK_BAGS, D), jnp.float32),
            pltpu.VMEM((CHUNK_BAGS, D), jnp.float32),
            pltpu.SemaphoreType.DMA,
            pltpu.SemaphoreType.DMA,
            pltpu.SemaphoreType.DMA,
            pltpu.SemaphoreType.DMA,
        ],
        compiler_params=pltpu.CompilerParams(use_tc_tiling_on_sc=False),
    )
    def emb_kernel(tab_hbm, idx_hbm, out_hbm,
                   idx_v0, idx_v1, rows_v0, rows_v1, out_v0, out_v1,
                   g0, g1, o0, o1):
        wid = lax.axis_index("s") * NC + lax.axis_index("c")

        def bag0_of(k):
            t = k // NCPT
            c = k % NCPT
            return t, t * bsub + wid * BAGS_PER_TW + c * CHUNK_BAGS

        def fire(k, idx_v, rows_v, gsem):
            t, bag0 = bag0_of(k)
            pltpu.sync_copy(idx_hbm.at[pl.ds(bag0 * L, IDX_PER_CHUNK)], idx_v)
            for j in range(IDX_ROWS):
                sl = pl.ds(j * 128, 128)
                pltpu.async_copy(
                    tab_hbm.at[t].at[idx_v.at[sl]], rows_v.at[sl], gsem
                )

        def drain_rows(rows_v, gsem):
            pltpu.make_async_copy(
                tab_hbm.at[0, pl.ds(0, IDX_PER_CHUNK)], rows_v, gsem
            ).wait()

        def accum_write(k, rows_v, out_v, osem):
            _, bag0 = bag0_of(k)

            @pl.when(k >= 2)
            def _():
                pltpu.make_async_copy(
                    out_hbm.at[pl.ds(0, CHUNK_BAGS)], out_v, osem
                ).wait()

            @pl.loop(0, CHUNK_BAGS)
            def _(w):
                base = w * L
                for d in range(D // 16):
                    sl = pl.ds(d * 16, 16)
                    a0 = rows_v[base, sl]
                    a1 = rows_v[base + 1, sl]
                    for i in range(2, L, 2):
                        a0 = a0 + rows_v[base + i, sl]
                        a1 = a1 + rows_v[base + i + 1, sl]
                    out_v[w, sl] = a0 + a1

            pltpu.async_copy(out_v, out_hbm.at[pl.ds(bag0, CHUNK_BAGS)], osem)

        fire(0, idx_v0, rows_v0, g0)

        @pl.loop(0, N_CHUNKS - 1)
        def _(k):
            @pl.when(k % 2 == 0)
            def _():
                fire(k + 1, idx_v1, rows_v1, g1)
                drain_rows(rows_v0, g0)
                accum_write(k, rows_v0, out_v0, o0)

            @pl.when(k % 2 == 1)
            def _():
                fire(k + 1, idx_v0, rows_v0, g0)
                drain_rows(rows_v1, g1)
                accum_write(k, rows_v1, out_v1, o1)

        drain_rows(rows_v1, g1)
        accum_write(N_CHUNKS - 1, rows_v1, out_v1, o1)

        pltpu.make_async_copy(
            out_hbm.at[pl.ds(0, CHUNK_BAGS)], out_v0, o0
        ).wait()
        pltpu.make_async_copy(
            out_hbm.at[pl.ds(0, CHUNK_BAGS)], out_v1, o1
        ).wait()

    return emb_kernel(tables, idx_flat)


BLK = 512


def _mlp_tc(dense_features, emb_flat, dense_W1, dense_b1, dense_W2, dense_b2,
            W1d, W1e, out_b1, out_W2, out_b2, nbatch):
    def body(df, emb, dW1, db1, dW2, db2, w1d, w1e, ob1, oW2, ob2, out):
        cdims = (((1,), (1,)), ((), ()))
        h1 = lax.dot_general(df[...], dW1[...], cdims,
                             preferred_element_type=jnp.float32)
        h1 = jnp.maximum(h1 + db1[...], 0.0)
        dx = lax.dot_general(h1, dW2[...], cdims,
                             preferred_element_type=jnp.float32)
        dx = jnp.maximum(dx + db2[...], 0.0)
        a = lax.dot_general(dx, w1d[...], cdims,
                            preferred_element_type=jnp.float32)
        w1e_all = w1e[...]
        for t in range(T):
            a = a + lax.dot_general(emb[t], w1e_all[:, t * D:(t + 1) * D],
                                    cdims, preferred_element_type=jnp.float32)
        h = jnp.maximum(a + ob1[...], 0.0)
        o = lax.dot_general(h, oW2[...], cdims,
                            preferred_element_type=jnp.float32)
        out[...] = jnp.maximum(o + ob2[0, 0], 0.0)

    F_E = T * D
    whole = lambda shape: pl.BlockSpec(shape, lambda i: (0, 0))
    return pl.pallas_call(
        body,
        grid=(nbatch // BLK,),
        in_specs=[
            pl.BlockSpec((BLK, DD), lambda i: (i, 0)),
            pl.BlockSpec((T, BLK, D), lambda i: (0, i, 0)),
            whole((DD, DD)),
            whole((1, DD)),
            whole((D, DD)),
            whole((1, D)),
            whole((H, D)),
            whole((H, F_E)),
            whole((1, H)),
            whole((128, H)),
            whole((1, 1)),
        ],
        out_specs=pl.BlockSpec((BLK, 128), lambda i: (i, 0)),
        out_shape=jax.ShapeDtypeStruct((nbatch, 128), jnp.float32),
    )(dense_features, emb_flat, dense_W1, dense_b1.reshape(1, DD),
      dense_W2, dense_b2.reshape(1, D), W1d, W1e, out_b1.reshape(1, H),
      jnp.zeros((128, H), jnp.float32).at[0].set(out_W2[0]),
      out_b2.reshape(1, 1))[:, :1]


def kernel(dense_features, sparse_features, dense_W1, dense_b1, dense_W2,
           dense_b2, tables, out_W1, out_b1, out_W2, out_b2):
    assert sparse_features.shape == (T, B, L)
    assert tables.shape == (T, V, D)

    W1d = out_W1[:, :D]
    W1e = out_W1[:, D:]

    NSPLIT = 2
    B2 = B // NSPLIT
    sparse_i32 = sparse_features.astype(jnp.int32)
    outs = []
    for h in range(NSPLIT):
        idx_h = sparse_i32[:, h * B2:(h + 1) * B2, :].reshape(-1)
        emb_h = _emb_bag_sc(tables, idx_h, T * B2)
        outs.append(_mlp_tc(dense_features[h * B2:(h + 1) * B2],
                            emb_h.reshape(T, B2, D), dense_W1, dense_b1,
                            dense_W2, dense_b2, W1d, W1e, out_b1, out_W2,
                            out_b2, B2))
    return jnp.concatenate(outs, axis=0)
